# baseline (speedup 1.0000x reference)
import jax
import jax.numpy as jnp
from jax import lax
from jax.experimental import pallas as pl
from jax.experimental.pallas import tpu as pltpu
from jax.experimental.pallas import tpu_sc as plsc

_D = 64
_V = 1_000_000
_B = 16384
_NW = 32
_BPW = _B // _NW
_HALF = _V // 2
_CB = 32768
_QW = _CB // 4
_GRID = (_V + _CB - 1) // _CB
_ROWS = _GRID * _QW


def _tc_transpose_body(x_ref, o_ref):
    eye = jnp.eye(_D, dtype=jnp.float32)
    for h2 in range(4):
        y = jax.lax.dot_general(
            x_ref[:, h2 * _QW:(h2 + 1) * _QW], eye, (((0,), (0,)), ((), ())),
            preferred_element_type=jnp.float32,
        )
        yb = y.astype(jnp.bfloat16).astype(jnp.float32)
        bits = jax.lax.bitcast_convert_type(yb, jnp.int32)
        lo = (bits[:, 0:32] >> 16) & 0xFFFF
        hi = bits[:, 32:64]
        o_ref[:, 32 * h2:32 * h2 + 32] = lo | hi


def _gather_body(wt_hbm, idx_hbm, out_hbm, idx_v, p_v, h_v, rows_v, out_v, sem):
    wid = lax.axis_index("s") * 2 + lax.axis_index("c")
    base_b = wid * _BPW
    pltpu.sync_copy(idx_hbm.at[pl.ds(base_b, _BPW)], idx_v)

    def split(g, _):
        iv = idx_v[pl.ds(g * 16, 16)]
        c = iv >> 15
        j = iv & 32767
        p_v[pl.ds(g * 16, 16)] = (c << 13) | (j & 8191)
        h_v[pl.ds(g * 16, 16)] = (iv >> 13) & 3
        return 0

    lax.fori_loop(0, _BPW // 16, split, 0)

    pltpu.make_async_copy(wt_hbm.at[p_v], rows_v, sem).start()
    pltpu.make_async_copy(wt_hbm.at[p_v], rows_v, sem).wait()

    mask_hi = jnp.full((16,), -65536, jnp.int32)

    def extract(g, _):
        hvec = h_v[pl.ds(g * 16, 16)]
        for l in range(16):
            row = g * 16 + l
            off = hvec[l] * 32
            for q in range(2):
                w = rows_v[row, pl.ds(off + 16 * q, 16)]
                lo = plsc.bitcast(w << 16, jnp.float32)
                hi = plsc.bitcast(w & mask_hi, jnp.float32)
                out_v[pl.ds(row * _D + 16 * q, 16)] = lo
                out_v[pl.ds(row * _D + 32 + 16 * q, 16)] = hi
        return 0

    lax.fori_loop(0, _BPW // 16, extract, 0)
    pltpu.sync_copy(out_v, out_hbm.at[pl.ds(base_b * _D, _BPW * _D)])


@jax.jit
def kernel(index, weights):
    idx32 = index.astype(jnp.int32)

    wt = pl.pallas_call(
        _tc_transpose_body,
        grid=(_GRID,),
        in_specs=[
            pl.BlockSpec((_D, _CB), lambda c: (0, c)),
        ],
        out_specs=pl.BlockSpec((_QW, 2 * _D), lambda c: (c, 0)),
        out_shape=jax.ShapeDtypeStruct((_ROWS, 2 * _D), jnp.int32),
    )(weights)

    gather = pl.kernel(
        _gather_body,
        out_type=jax.ShapeDtypeStruct((_B * _D,), jnp.float32),
        mesh=plsc.VectorSubcoreMesh(core_axis_name="c", subcore_axis_name="s"),
        compiler_params=pltpu.CompilerParams(needs_layout_passes=False),
        scratch_types=[
            pltpu.VMEM((_BPW,), jnp.int32),
            pltpu.VMEM((_BPW,), jnp.int32),
            pltpu.VMEM((_BPW,), jnp.int32),
            pltpu.VMEM((_BPW, 2 * _D), jnp.int32),
            pltpu.VMEM((_BPW * _D,), jnp.float32),
            pltpu.SemaphoreType.DMA,
        ],
    )

    out = gather(wt, idx32)
    return out.reshape(_B, _D)


# speedup vs baseline: 1.3879x; 1.3879x over previous
import jax
import jax.numpy as jnp
from jax import lax
from jax.experimental import pallas as pl
from jax.experimental.pallas import tpu as pltpu
from jax.experimental.pallas import tpu_sc as plsc

_D = 64
_V = 1_000_000
_B = 16384
_NW = 32
_BPW = _B // _NW
_HALF = _V // 2
_CB = 32768
_RB = _CB // 2
_GRID = (_V + _CB - 1) // _CB
_ROWS = _GRID * _RB


def _tc_transpose_body(x_ref, o_ref):
    o_ref[:, 0:_D] = x_ref[:, 0:_RB].T
    o_ref[:, _D:2 * _D] = x_ref[:, _RB:_CB].T


def _gather_body(wt_hbm, idx_hbm, out_hbm, idx_v, p_v, h_v, rows_v, out_v, sem):
    wid = lax.axis_index("s") * 2 + lax.axis_index("c")
    base_b = wid * _BPW
    pltpu.sync_copy(idx_hbm.at[pl.ds(base_b, _BPW)], idx_v)

    def split(g, _):
        iv = idx_v[pl.ds(g * 16, 16)]
        c = iv >> 15
        j = iv & 32767
        p_v[pl.ds(g * 16, 16)] = (c << 14) | (j & 16383)
        h_v[pl.ds(g * 16, 16)] = (iv >> 14) & 1
        return 0

    lax.fori_loop(0, _BPW // 16, split, 0)

    pltpu.make_async_copy(wt_hbm.at[p_v], rows_v, sem).start()
    pltpu.make_async_copy(wt_hbm.at[p_v], rows_v, sem).wait()

    def extract(g, _):
        hvec = h_v[pl.ds(g * 16, 16)]
        for l in range(16):
            row = g * 16 + l
            off = hvec[l] * _D
            for j in range(4):
                out_v[pl.ds(row * _D + 16 * j, 16)] = rows_v[row, pl.ds(off + 16 * j, 16)]
        return 0

    lax.fori_loop(0, _BPW // 16, extract, 0)
    pltpu.sync_copy(out_v, out_hbm.at[pl.ds(base_b * _D, _BPW * _D)])


@jax.jit
def kernel(index, weights):
    idx32 = index.astype(jnp.int32)

    wt = pl.pallas_call(
        _tc_transpose_body,
        grid=(_GRID,),
        in_specs=[
            pl.BlockSpec((_D, _CB), lambda c: (0, c)),
        ],
        out_specs=pl.BlockSpec((_RB, 2 * _D), lambda c: (c, 0)),
        out_shape=jax.ShapeDtypeStruct((_ROWS, 2 * _D), jnp.float32),
    )(weights)

    gather = pl.kernel(
        _gather_body,
        out_type=jax.ShapeDtypeStruct((_B * _D,), jnp.float32),
        mesh=plsc.VectorSubcoreMesh(core_axis_name="c", subcore_axis_name="s"),
        compiler_params=pltpu.CompilerParams(needs_layout_passes=False),
        scratch_types=[
            pltpu.VMEM((_BPW,), jnp.int32),
            pltpu.VMEM((_BPW,), jnp.int32),
            pltpu.VMEM((_BPW,), jnp.int32),
            pltpu.VMEM((_BPW, 2 * _D), jnp.float32),
            pltpu.VMEM((_BPW * _D,), jnp.float32),
            pltpu.SemaphoreType.DMA,
        ],
    )

    out = gather(wt, idx32)
    return out.reshape(_B, _D)


# speedup vs baseline: 1.3899x; 1.0014x over previous
import jax
import jax.numpy as jnp
from jax import lax
from jax.experimental import pallas as pl
from jax.experimental.pallas import tpu as pltpu
from jax.experimental.pallas import tpu_sc as plsc

_D = 64
_V = 1_000_000
_B = 16384
_NW = 32
_BPW = _B // _NW
_HALF = _V // 2
_CB = 32768
_RB = _CB // 2
_GRID = (_V + _CB - 1) // _CB
_ROWS = _GRID * _RB


def _tc_transpose_body(x_ref, o_ref):
    o_ref[:, 0:_D] = x_ref[:, 0:_RB].T
    o_ref[:, _D:2 * _D] = x_ref[:, _RB:_CB].T


def _gather_body(wt_hbm, idx_hbm, out_hbm, idx_v, p_v, h_v, rows_v, out_v,
                 s0, s1, s2, s3, wsem):
    sems = (s0, s1, s2, s3)
    wid = lax.axis_index("s") * 2 + lax.axis_index("c")
    base_b = wid * _BPW
    pltpu.sync_copy(idx_hbm.at[pl.ds(base_b, _BPW)], idx_v)

    def split(g, _):
        iv = idx_v[pl.ds(g * 16, 16)]
        c = iv >> 15
        j = iv & 32767
        p_v[pl.ds(g * 16, 16)] = (c << 14) | (j & 16383)
        h_v[pl.ds(g * 16, 16)] = (iv >> 14) & 1
        return 0

    lax.fori_loop(0, _BPW // 16, split, 0)

    for t in range(4):
        pltpu.make_async_copy(
            wt_hbm.at[p_v.at[pl.ds(128 * t, 128)]],
            rows_v.at[pl.ds(128 * t, 128)],
            sems[t],
        ).start()

    for t in range(4):
        pltpu.make_async_copy(
            wt_hbm.at[p_v.at[pl.ds(128 * t, 128)]],
            rows_v.at[pl.ds(128 * t, 128)],
            sems[t],
        ).wait()

        def extract(g, _, t=t):
            hvec = h_v[pl.ds(128 * t + g * 16, 16)]
            for l in range(16):
                row = 128 * t + g * 16 + l
                off = hvec[l] * _D
                for j in range(4):
                    out_v[pl.ds(row * _D + 16 * j, 16)] = rows_v[row, pl.ds(off + 16 * j, 16)]
            return 0

        lax.fori_loop(0, 8, extract, 0)
        pltpu.make_async_copy(
            out_v.at[pl.ds(128 * t * _D, 128 * _D)],
            out_hbm.at[pl.ds((base_b + 128 * t) * _D, 128 * _D)],
            wsem,
        ).start()

    for t in range(4):
        pltpu.make_async_copy(
            out_v.at[pl.ds(128 * t * _D, 128 * _D)],
            out_hbm.at[pl.ds((base_b + 128 * t) * _D, 128 * _D)],
            wsem,
        ).wait()


@jax.jit
def kernel(index, weights):
    idx32 = index.astype(jnp.int32)

    wt = pl.pallas_call(
        _tc_transpose_body,
        grid=(_GRID,),
        in_specs=[
            pl.BlockSpec((_D, _CB), lambda c: (0, c)),
        ],
        out_specs=pl.BlockSpec((_RB, 2 * _D), lambda c: (c, 0)),
        out_shape=jax.ShapeDtypeStruct((_ROWS, 2 * _D), jnp.float32),
    )(weights)

    gather = pl.kernel(
        _gather_body,
        out_type=jax.ShapeDtypeStruct((_B * _D,), jnp.float32),
        mesh=plsc.VectorSubcoreMesh(core_axis_name="c", subcore_axis_name="s"),
        compiler_params=pltpu.CompilerParams(needs_layout_passes=False),
        scratch_types=[
            pltpu.VMEM((_BPW,), jnp.int32),
            pltpu.VMEM((_BPW,), jnp.int32),
            pltpu.VMEM((_BPW,), jnp.int32),
            pltpu.VMEM((_BPW, 2 * _D), jnp.float32),
            pltpu.VMEM((_BPW * _D,), jnp.float32),
            pltpu.SemaphoreType.DMA,
            pltpu.SemaphoreType.DMA,
            pltpu.SemaphoreType.DMA,
            pltpu.SemaphoreType.DMA,
            pltpu.SemaphoreType.DMA,
        ],
    )

    out = gather(wt, idx32)
    return out.reshape(_B, _D)


# speedup vs baseline: 1.3910x; 1.0008x over previous
import jax
import jax.numpy as jnp
from jax import lax
from jax.experimental import pallas as pl
from jax.experimental.pallas import tpu as pltpu
from jax.experimental.pallas import tpu_sc as plsc

_D = 64
_V = 1_000_000
_B = 16384
_NW = 32
_BPW = _B // _NW
_CB = 32768
_RB = _CB // 2
_GRID = (_V + _CB - 1) // _CB
_ROWS = _GRID * _RB


def _tc_transpose_body(x_ref, o_ref):
    o_ref[:, 0:_D] = x_ref[:, 0:_RB].T
    o_ref[:, _D:2 * _D] = x_ref[:, _RB:_CB].T


def _gather_body(wt_hbm, idx_hbm, out_hbm, idx_v, p_v, h_v, rows_v, out_v,
                 s0, s1, s2, s3, wsem):
    sems = (s0, s1, s2, s3)
    wid = lax.axis_index("s") * 2 + lax.axis_index("c")
    base_b = wid * _BPW
    pltpu.sync_copy(idx_hbm.at[pl.ds(base_b, _BPW)], idx_v)

    def split(g, _):
        iv = idx_v[pl.ds(g * 16, 16)]
        c = iv >> 15
        j = iv & 32767
        p_v[pl.ds(g * 16, 16)] = (c << 14) | (j & 16383)
        h_v[pl.ds(g * 16, 16)] = (iv >> 14) & 1
        return 0

    lax.fori_loop(0, _BPW // 16, split, 0)

    for t in range(4):
        pltpu.make_async_copy(
            wt_hbm.at[p_v.at[pl.ds(128 * t, 128)]],
            rows_v.at[pl.ds(128 * t, 128)],
            sems[t],
        ).start()

    for t in range(4):
        pltpu.make_async_copy(
            wt_hbm.at[p_v.at[pl.ds(128 * t, 128)]],
            rows_v.at[pl.ds(128 * t, 128)],
            sems[t],
        ).wait()

        def extract(g, _, t=t):
            hvec = h_v[pl.ds(128 * t + g * 16, 16)]
            for l in range(16):
                row = 128 * t + g * 16 + l
                off = hvec[l] * _D
                for j in range(4):
                    out_v[pl.ds(row * _D + 16 * j, 16)] = rows_v[row, pl.ds(off + 16 * j, 16)]
            return 0

        lax.fori_loop(0, 8, extract, 0)
        pltpu.make_async_copy(
            out_v.at[pl.ds(128 * t * _D, 128 * _D)],
            out_hbm.at[pl.ds((base_b + 128 * t) * _D, 128 * _D)],
            wsem,
        ).start()

    for t in range(4):
        pltpu.make_async_copy(
            out_v.at[pl.ds(128 * t * _D, 128 * _D)],
            out_hbm.at[pl.ds((base_b + 128 * t) * _D, 128 * _D)],
            wsem,
        ).wait()


@jax.jit
def kernel(index, weights):
    idx32 = index.astype(jnp.int32)

    wt = pl.pallas_call(
        _tc_transpose_body,
        grid=(_GRID,),
        in_specs=[
            pl.BlockSpec((_D, _CB), lambda c: (0, c)),
        ],
        out_specs=pl.BlockSpec((_RB, 2 * _D), lambda c: (c, 0)),
        out_shape=jax.ShapeDtypeStruct((_ROWS, 2 * _D), jnp.float32),
    )(weights)

    gather = pl.kernel(
        _gather_body,
        out_type=jax.ShapeDtypeStruct((_B * _D,), jnp.float32),
        mesh=plsc.VectorSubcoreMesh(core_axis_name="c", subcore_axis_name="s"),
        compiler_params=pltpu.CompilerParams(needs_layout_passes=False),
        scratch_types=[
            pltpu.VMEM((_BPW,), jnp.int32),
            pltpu.VMEM((_BPW,), jnp.int32),
            pltpu.VMEM((_BPW,), jnp.int32),
            pltpu.VMEM((_BPW, 2 * _D), jnp.float32),
            pltpu.VMEM((_BPW * _D,), jnp.float32),
            pltpu.SemaphoreType.DMA,
            pltpu.SemaphoreType.DMA,
            pltpu.SemaphoreType.DMA,
            pltpu.SemaphoreType.DMA,
            pltpu.SemaphoreType.DMA,
        ],
    )

    out = gather(wt, idx32)
    return out.reshape(_B, _D)
